# baseline (speedup 1.0000x reference)
import functools

import jax
import jax.numpy as jnp
from jax import lax
from jax.experimental import pallas as pl
from jax.experimental.pallas import tpu as pltpu
from jax.experimental.pallas import tpu_sc as plsc

_B = 16384
_H = 50
_D = 32
_V = 1000000
_NW = 32
_NB = _B // 128
_NT_FULL = (_H // 8) * _NB
_NT_PART = _NB
_FLAT_N = _B * _H
_NBLK = _FLAT_N // 128
_MESH = plsc.VectorSubcoreMesh(core_axis_name="c", subcore_axis_name="s")


def _wid():
    return lax.axis_index("s") * 2 + lax.axis_index("c")


@functools.partial(
    pl.kernel,
    mesh=_MESH,
    out_type=jax.ShapeDtypeStruct((_FLAT_N,), jnp.int32),
    scratch_types=[
        pltpu.VMEM((8, 128), jnp.int32),
        pltpu.VMEM((2, 128), jnp.int32),
        pltpu.VMEM((1024,), jnp.int32),
    ],
    compiler_params=pltpu.CompilerParams(
        use_tc_tiling_on_sc=True, needs_layout_passes=False),
)
def _stage_a(inp2, flat, ibuf, pbuf, fbuf):
    w = _wid()

    def full_tile(ci, carry):
        t = ci * _NW + w
        gh = t // _NB
        cb = t % _NB
        pltpu.sync_copy(
            inp2.at[pl.ds(gh * 8, 8), pl.ds(cb * 128, 128)], ibuf)
        for k in range(8):
            for j in range(8):
                fbuf[pl.ds(k * 128 + j * 16, 16)] = ibuf[k, pl.ds(j * 16, 16)]
        pltpu.sync_copy(fbuf, flat.at[pl.ds(t * 1024, 1024)])
        return carry

    lax.fori_loop(0, _NT_FULL // _NW, full_tile, 0)

    def part_tile(ci, carry):
        t = ci * _NW + w
        pltpu.sync_copy(
            inp2.at[pl.ds(48, 2), pl.ds(t * 128, 128)], pbuf)
        for k in range(2):
            for j in range(8):
                fbuf[pl.ds(k * 128 + j * 16, 16)] = pbuf[k, pl.ds(j * 16, 16)]
        pltpu.sync_copy(
            fbuf.at[pl.ds(0, 256)],
            flat.at[pl.ds(_NT_FULL * 1024 + t * 256, 256)])
        return carry

    lax.fori_loop(0, _NT_PART // _NW, part_tile, 0)


@functools.partial(
    pl.kernel,
    mesh=_MESH,
    out_type=jax.ShapeDtypeStruct((_FLAT_N * _D,), jnp.float32),
    scratch_types=[
        pltpu.VMEM((128,), jnp.int32),
        pltpu.VMEM((128,), jnp.int32),
        pltpu.VMEM((128, _D), jnp.float32),
        pltpu.VMEM((128, _D), jnp.float32),
        pltpu.VMEM((4096,), jnp.float32),
        pltpu.VMEM((4096,), jnp.float32),
        pltpu.SemaphoreType.DMA,
        pltpu.SemaphoreType.DMA,
        pltpu.SemaphoreType.DMA,
        pltpu.SemaphoreType.DMA,
        pltpu.SemaphoreType.DMA,
        pltpu.SemaphoreType.DMA,
    ],
    compiler_params=pltpu.CompilerParams(
        use_tc_tiling_on_sc=False, needs_layout_passes=False),
)
def _stage_b(flat, rv, g1d, idx0, idx1, gb0, gb1, ob0, ob1,
             is0, is1, gs0, gs1, os0, os1):
    w = _wid()
    idxb = [idx0, idx1]
    gbuf = [gb0, gb1]
    obuf = [ob0, ob1]
    isem = [is0, is1]
    gsem = [gs0, gs1]
    osem = [os0, os1]
    lane16 = lax.iota(jnp.int32, 16)
    nblk = _NBLK // _NW

    def idx_wait(b, i):
        pltpu.make_async_copy(
            flat.at[pl.ds(i * 128, 128)], idxb[b], isem[b]).wait()

    def gather_start(b):
        pltpu.async_copy(rv.at[idxb[b]], gbuf[b], gsem[b])

    def gather_wait(b):
        pltpu.make_async_copy(
            rv.at[idxb[b]], gbuf[b], gsem[b]).wait()

    def transpose(b):
        for d in range(32):
            dvec = lane16 * 0 + d
            for j in range(8):
                v = plsc.load_gather(gbuf[b], [lane16 + j * 16, dvec])
                obuf[b][pl.ds(d * 128 + j * 16, 16)] = v

    def out_start(b, i):
        pltpu.async_copy(obuf[b], g1d.at[pl.ds(i * 4096, 4096)], osem[b])

    def out_wait(b, i):
        pltpu.make_async_copy(
            obuf[b], g1d.at[pl.ds(i * 4096, 4096)], osem[b]).wait()

    for b in range(2):
        pltpu.async_copy(
            flat.at[pl.ds((b * _NW + w) * 128, 128)], idxb[b], isem[b])
    idx_wait(0, w)
    gather_start(0)

    def step(ci, b, pb):
        i_next = (ci + 1) * _NW + w
        i_cur = ci * _NW + w
        idx_wait(b, i_next)

        def _wait_ob():
            out_wait(b, (ci - 1) * _NW + w)

        if isinstance(ci, int):
            if ci >= 1:
                _wait_ob()
        else:
            pl.when(ci >= 1)(_wait_ob)

        gather_start(b)
        gather_wait(pb)

        def _prefetch():
            pltpu.async_copy(
                flat.at[pl.ds(((ci + 2) * _NW + w) * 128, 128)],
                idxb[pb], isem[pb])

        if isinstance(ci, int):
            if ci + 2 < nblk:
                _prefetch()
        else:
            pl.when(ci + 2 < nblk)(_prefetch)

        transpose(pb)
        out_start(pb, i_cur)

    def body(o, carry):
        for p in range(2):
            ci = o * 2 + p
            step(ci, (p + 1) % 2, p)
        return carry

    lax.fori_loop(0, (nblk - 2) // 2, body, 0)
    step(nblk - 2, (nblk - 1) % 2, (nblk - 2) % 2)

    lb = (nblk - 1) % 2
    li = (nblk - 1) * _NW + w
    gather_wait(lb)
    transpose(lb)
    out_start(lb, li)
    out_wait(lb, li)
    out_wait((nblk - 2) % 2, (nblk - 2) * _NW + w)


@functools.partial(
    pl.kernel,
    mesh=_MESH,
    out_type=jax.ShapeDtypeStruct((_H, _D, _B), jnp.float32),
    scratch_types=[
        pltpu.VMEM((4096,), jnp.float32),
        pltpu.VMEM((32, 128), jnp.float32),
    ],
    compiler_params=pltpu.CompilerParams(
        use_tc_tiling_on_sc=True, needs_layout_passes=False),
)
def _stage_c(g1d, out50, dbuf, cbuf):
    w = _wid()
    nblk = _NBLK // _NW

    def block(ci, carry):
        i = ci * _NW + w
        for d in range(32):
            pltpu.sync_copy(
                g1d.at[pl.ds(i * 4096 + d * 128, 128)], cbuf.at[d])
        in_full = i < _NT_FULL * 8
        t = jnp.where(in_full, i // 8, 0)
        h_full = (t // _NB) * 8 + (i % 8)
        cb_full = t % _NB
        r = i - _NT_FULL * 8
        h_part = 48 + (r % 2)
        cb_part = r // 2
        h = jnp.where(in_full, h_full, h_part)
        cb = jnp.where(in_full, cb_full, cb_part)
        pltpu.sync_copy(cbuf, out50.at[h, :, pl.ds(cb * 128, 128)])
        return carry

    lax.fori_loop(0, nblk, block, 0)


def kernel(input, table):
    inp2 = input.T
    flat = _stage_a(inp2)
    g1d = _stage_b(flat, table)
    out50 = _stage_c(g1d)
    return out50.transpose(2, 0, 1)


# speedup vs baseline: 2.8364x; 2.8364x over previous
import functools

import jax
import jax.numpy as jnp
from jax import lax
from jax.experimental import pallas as pl
from jax.experimental.pallas import tpu as pltpu
from jax.experimental.pallas import tpu_sc as plsc

_B = 16384
_H = 50
_D = 32
_V = 1000000
_NW = 32
_NB = _B // 128
_NT_FULL = (_H // 8) * _NB
_NT_PART = _NB
_FLAT_N = _B * _H
_NBLK = _FLAT_N // 128
_MESH = plsc.VectorSubcoreMesh(core_axis_name="c", subcore_axis_name="s")


def _wid():
    return lax.axis_index("s") * 2 + lax.axis_index("c")


@functools.partial(
    pl.kernel,
    mesh=_MESH,
    out_type=jax.ShapeDtypeStruct((_FLAT_N,), jnp.int32),
    scratch_types=[
        pltpu.VMEM((8, 128), jnp.int32),
        pltpu.VMEM((2, 128), jnp.int32),
        pltpu.VMEM((1024,), jnp.int32),
    ],
    compiler_params=pltpu.CompilerParams(
        use_tc_tiling_on_sc=True, needs_layout_passes=False),
)
def _stage_a(inp2, flat, ibuf, pbuf, fbuf):
    w = _wid()

    def full_tile(ci, carry):
        t = ci * _NW + w
        gh = t // _NB
        cb = t % _NB
        pltpu.sync_copy(
            inp2.at[pl.ds(gh * 8, 8), pl.ds(cb * 128, 128)], ibuf)
        for k in range(8):
            for j in range(8):
                fbuf[pl.ds(k * 128 + j * 16, 16)] = ibuf[k, pl.ds(j * 16, 16)]
        pltpu.sync_copy(fbuf, flat.at[pl.ds(t * 1024, 1024)])
        return carry

    lax.fori_loop(0, _NT_FULL // _NW, full_tile, 0)

    def part_tile(ci, carry):
        t = ci * _NW + w
        pltpu.sync_copy(
            inp2.at[pl.ds(48, 2), pl.ds(t * 128, 128)], pbuf)
        for k in range(2):
            for j in range(8):
                fbuf[pl.ds(k * 128 + j * 16, 16)] = pbuf[k, pl.ds(j * 16, 16)]
        pltpu.sync_copy(
            fbuf.at[pl.ds(0, 256)],
            flat.at[pl.ds(_NT_FULL * 1024 + t * 256, 256)])
        return carry

    lax.fori_loop(0, _NT_PART // _NW, part_tile, 0)


@functools.partial(
    pl.kernel,
    mesh=_MESH,
    out_type=jax.ShapeDtypeStruct((_FLAT_N * _D,), jnp.float32),
    scratch_types=[
        pltpu.VMEM((128,), jnp.int32),
        pltpu.VMEM((128,), jnp.int32),
        pltpu.VMEM((128, _D), jnp.float32),
        pltpu.VMEM((128, _D), jnp.float32),
        pltpu.VMEM((4096,), jnp.float32),
        pltpu.VMEM((4096,), jnp.float32),
        pltpu.SemaphoreType.DMA,
        pltpu.SemaphoreType.DMA,
        pltpu.SemaphoreType.DMA,
        pltpu.SemaphoreType.DMA,
        pltpu.SemaphoreType.DMA,
        pltpu.SemaphoreType.DMA,
    ],
    compiler_params=pltpu.CompilerParams(
        use_tc_tiling_on_sc=False, needs_layout_passes=False),
)
def _stage_b(flat, rv, g1d, idx0, idx1, gb0, gb1, ob0, ob1,
             is0, is1, gs0, gs1, os0, os1):
    w = _wid()
    idxb = [idx0, idx1]
    gbuf = [gb0, gb1]
    obuf = [ob0, ob1]
    isem = [is0, is1]
    gsem = [gs0, gs1]
    osem = [os0, os1]
    lane16 = lax.iota(jnp.int32, 16)
    nblk = _NBLK // _NW

    def idx_wait(b, i):
        pltpu.make_async_copy(
            flat.at[pl.ds(i * 128, 128)], idxb[b], isem[b]).wait()

    def gather_start(b):
        pltpu.async_copy(rv.at[idxb[b]], gbuf[b], gsem[b])

    def gather_wait(b):
        pltpu.make_async_copy(
            rv.at[idxb[b]], gbuf[b], gsem[b]).wait()

    def transpose(b):
        for d in range(32):
            dvec = lane16 * 0 + d
            for j in range(8):
                v = plsc.load_gather(gbuf[b], [lane16 + j * 16, dvec])
                obuf[b][pl.ds(d * 128 + j * 16, 16)] = v

    def out_start(b, i):
        pltpu.async_copy(obuf[b], g1d.at[pl.ds(i * 4096, 4096)], osem[b])

    def out_wait(b, i):
        pltpu.make_async_copy(
            obuf[b], g1d.at[pl.ds(i * 4096, 4096)], osem[b]).wait()

    for b in range(2):
        pltpu.async_copy(
            flat.at[pl.ds((b * _NW + w) * 128, 128)], idxb[b], isem[b])
    idx_wait(0, w)
    gather_start(0)

    def step(ci, b, pb):
        i_next = (ci + 1) * _NW + w
        i_cur = ci * _NW + w
        idx_wait(b, i_next)

        def _wait_ob():
            out_wait(b, (ci - 1) * _NW + w)

        if isinstance(ci, int):
            if ci >= 1:
                _wait_ob()
        else:
            pl.when(ci >= 1)(_wait_ob)

        gather_start(b)
        gather_wait(pb)

        def _prefetch():
            pltpu.async_copy(
                flat.at[pl.ds(((ci + 2) * _NW + w) * 128, 128)],
                idxb[pb], isem[pb])

        if isinstance(ci, int):
            if ci + 2 < nblk:
                _prefetch()
        else:
            pl.when(ci + 2 < nblk)(_prefetch)

        transpose(pb)
        out_start(pb, i_cur)

    def body(o, carry):
        for p in range(2):
            ci = o * 2 + p
            step(ci, (p + 1) % 2, p)
        return carry

    lax.fori_loop(0, (nblk - 2) // 2, body, 0)
    step(nblk - 2, (nblk - 1) % 2, (nblk - 2) % 2)

    lb = (nblk - 1) % 2
    li = (nblk - 1) * _NW + w
    gather_wait(lb)
    transpose(lb)
    out_start(lb, li)
    out_wait(lb, li)
    out_wait((nblk - 2) % 2, (nblk - 2) * _NW + w)


@functools.partial(
    pl.kernel,
    mesh=_MESH,
    out_type=jax.ShapeDtypeStruct((_H, _D, _B), jnp.float32),
    scratch_types=[
        pltpu.VMEM((4096,), jnp.float32),
        pltpu.VMEM((32, 128), jnp.float32),
    ],
    compiler_params=pltpu.CompilerParams(
        use_tc_tiling_on_sc=True, needs_layout_passes=False),
)
def _stage_c(g1d, out50, dbuf, cbuf):
    w = _wid()
    nblk = _NBLK // _NW

    def block(ci, carry):
        i = ci * _NW + w
        pltpu.sync_copy(g1d.at[pl.ds(i * 4096, 4096)], dbuf)
        for d in range(32):
            for j in range(8):
                cbuf[d, pl.ds(j * 16, 16)] = dbuf[pl.ds(d * 128 + j * 16, 16)]
        in_full = i < _NT_FULL * 8
        t = jnp.where(in_full, i // 8, 0)
        h_full = (t // _NB) * 8 + (i % 8)
        cb_full = t % _NB
        r = i - _NT_FULL * 8
        h_part = 48 + (r % 2)
        cb_part = r // 2
        h = jnp.where(in_full, h_full, h_part)
        cb = jnp.where(in_full, cb_full, cb_part)
        pltpu.sync_copy(cbuf, out50.at[h, :, pl.ds(cb * 128, 128)])
        return carry

    lax.fori_loop(0, nblk, block, 0)


def kernel(input, table):
    inp2 = input.T
    flat = _stage_a(inp2)
    g1d = _stage_b(flat, table)
    out50 = _stage_c(g1d)
    return out50.transpose(2, 0, 1)


# speedup vs baseline: 3.2982x; 1.1628x over previous
import functools

import jax
import jax.numpy as jnp
from jax import lax
from jax.experimental import pallas as pl
from jax.experimental.pallas import tpu as pltpu
from jax.experimental.pallas import tpu_sc as plsc

_B = 16384
_H = 50
_D = 32
_V = 1000000
_NW = 32
_NB = _B // 128
_NT_FULL = (_H // 8) * _NB
_NT_PART = _NB
_FLAT_N = _B * _H
_NBLK = _FLAT_N // 128
_MESH = plsc.VectorSubcoreMesh(core_axis_name="c", subcore_axis_name="s")


def _wid():
    return lax.axis_index("s") * 2 + lax.axis_index("c")


@functools.partial(
    pl.kernel,
    mesh=_MESH,
    out_type=jax.ShapeDtypeStruct((_FLAT_N,), jnp.int32),
    scratch_types=[
        pltpu.VMEM((8, 128), jnp.int32),
        pltpu.VMEM((2, 128), jnp.int32),
        pltpu.VMEM((1024,), jnp.int32),
    ],
    compiler_params=pltpu.CompilerParams(
        use_tc_tiling_on_sc=True, needs_layout_passes=False),
)
def _stage_a(inp2, flat, ibuf, pbuf, fbuf):
    w = _wid()

    def full_tile(ci, carry):
        t = ci * _NW + w
        gh = t // _NB
        cb = t % _NB
        pltpu.sync_copy(
            inp2.at[pl.ds(gh * 8, 8), pl.ds(cb * 128, 128)], ibuf)
        for k in range(8):
            for j in range(8):
                fbuf[pl.ds(k * 128 + j * 16, 16)] = ibuf[k, pl.ds(j * 16, 16)]
        pltpu.sync_copy(fbuf, flat.at[pl.ds(t * 1024, 1024)])
        return carry

    lax.fori_loop(0, _NT_FULL // _NW, full_tile, 0)

    def part_tile(ci, carry):
        t = ci * _NW + w
        pltpu.sync_copy(
            inp2.at[pl.ds(48, 2), pl.ds(t * 128, 128)], pbuf)
        for k in range(2):
            for j in range(8):
                fbuf[pl.ds(k * 128 + j * 16, 16)] = pbuf[k, pl.ds(j * 16, 16)]
        pltpu.sync_copy(
            fbuf.at[pl.ds(0, 256)],
            flat.at[pl.ds(_NT_FULL * 1024 + t * 256, 256)])
        return carry

    lax.fori_loop(0, _NT_PART // _NW, part_tile, 0)


@functools.partial(
    pl.kernel,
    mesh=_MESH,
    out_type=jax.ShapeDtypeStruct((_FLAT_N * _D,), jnp.float32),
    scratch_types=[
        pltpu.VMEM((128,), jnp.int32),
        pltpu.VMEM((128,), jnp.int32),
        pltpu.VMEM((128, _D), jnp.float32),
        pltpu.VMEM((128, _D), jnp.float32),
        pltpu.VMEM((4096,), jnp.float32),
        pltpu.VMEM((4096,), jnp.float32),
        pltpu.SemaphoreType.DMA,
        pltpu.SemaphoreType.DMA,
        pltpu.SemaphoreType.DMA,
        pltpu.SemaphoreType.DMA,
        pltpu.SemaphoreType.DMA,
        pltpu.SemaphoreType.DMA,
    ],
    compiler_params=pltpu.CompilerParams(
        use_tc_tiling_on_sc=False, needs_layout_passes=False),
)
def _stage_b(flat, rv, g1d, idx0, idx1, gb0, gb1, ob0, ob1,
             is0, is1, gs0, gs1, os0, os1):
    w = _wid()
    idxb = [idx0, idx1]
    gbuf = [gb0, gb1]
    obuf = [ob0, ob1]
    isem = [is0, is1]
    gsem = [gs0, gs1]
    osem = [os0, os1]
    lane128 = lax.iota(jnp.int32, 16) * 128
    c_lo = [lane128 + r for r in range(8)]
    c_hi = [lane128 + (2048 + r) for r in range(8)]
    nblk = _NBLK // _NW

    def idx_wait(b, i):
        pltpu.make_async_copy(
            flat.at[pl.ds(i * 128, 128)], idxb[b], isem[b]).wait()

    def gather_start(b):
        pltpu.async_copy(rv.at[idxb[b]], gbuf[b], gsem[b])

    def gather_wait(b):
        pltpu.make_async_copy(
            rv.at[idxb[b]], gbuf[b], gsem[b]).wait()

    def transpose(b):
        for l in range(128):
            l8 = (l // 8) * 8
            r = l - l8
            ref = obuf[b].at[pl.ds(l8, 4096 - l8)]
            v0 = gbuf[b][l, pl.ds(0, 16)]
            plsc.store_scatter(ref, [c_lo[r]], v0)
            v1 = gbuf[b][l, pl.ds(16, 16)]
            plsc.store_scatter(ref, [c_hi[r]], v1)

    def out_start(b, i):
        pltpu.async_copy(obuf[b], g1d.at[pl.ds(i * 4096, 4096)], osem[b])

    def out_wait(b, i):
        pltpu.make_async_copy(
            obuf[b], g1d.at[pl.ds(i * 4096, 4096)], osem[b]).wait()

    for b in range(2):
        pltpu.async_copy(
            flat.at[pl.ds((b * _NW + w) * 128, 128)], idxb[b], isem[b])
    idx_wait(0, w)
    gather_start(0)

    def step(ci, b, pb):
        i_next = (ci + 1) * _NW + w
        i_cur = ci * _NW + w
        idx_wait(b, i_next)

        def _wait_ob():
            out_wait(b, (ci - 1) * _NW + w)

        if isinstance(ci, int):
            if ci >= 1:
                _wait_ob()
        else:
            pl.when(ci >= 1)(_wait_ob)

        gather_start(b)
        gather_wait(pb)

        def _prefetch():
            pltpu.async_copy(
                flat.at[pl.ds(((ci + 2) * _NW + w) * 128, 128)],
                idxb[pb], isem[pb])

        if isinstance(ci, int):
            if ci + 2 < nblk:
                _prefetch()
        else:
            pl.when(ci + 2 < nblk)(_prefetch)

        transpose(pb)
        out_start(pb, i_cur)

    def body(o, carry):
        for p in range(2):
            ci = o * 2 + p
            step(ci, (p + 1) % 2, p)
        return carry

    lax.fori_loop(0, (nblk - 2) // 2, body, 0)
    step(nblk - 2, (nblk - 1) % 2, (nblk - 2) % 2)

    lb = (nblk - 1) % 2
    li = (nblk - 1) * _NW + w
    gather_wait(lb)
    transpose(lb)
    out_start(lb, li)
    out_wait(lb, li)
    out_wait((nblk - 2) % 2, (nblk - 2) * _NW + w)


@functools.partial(
    pl.kernel,
    mesh=_MESH,
    out_type=jax.ShapeDtypeStruct((_H, _D, _B), jnp.float32),
    scratch_types=[
        pltpu.VMEM((4096,), jnp.float32),
        pltpu.VMEM((32, 128), jnp.float32),
    ],
    compiler_params=pltpu.CompilerParams(
        use_tc_tiling_on_sc=True, needs_layout_passes=False),
)
def _stage_c(g1d, out50, dbuf, cbuf):
    w = _wid()
    nblk = _NBLK // _NW

    def block(ci, carry):
        i = ci * _NW + w
        pltpu.sync_copy(g1d.at[pl.ds(i * 4096, 4096)], dbuf)
        for d in range(32):
            for j in range(8):
                cbuf[d, pl.ds(j * 16, 16)] = dbuf[pl.ds(d * 128 + j * 16, 16)]
        in_full = i < _NT_FULL * 8
        t = jnp.where(in_full, i // 8, 0)
        h_full = (t // _NB) * 8 + (i % 8)
        cb_full = t % _NB
        r = i - _NT_FULL * 8
        h_part = 48 + (r % 2)
        cb_part = r // 2
        h = jnp.where(in_full, h_full, h_part)
        cb = jnp.where(in_full, cb_full, cb_part)
        pltpu.sync_copy(cbuf, out50.at[h, :, pl.ds(cb * 128, 128)])
        return carry

    lax.fori_loop(0, nblk, block, 0)


def kernel(input, table):
    inp2 = input.T
    flat = _stage_a(inp2)
    g1d = _stage_b(flat, table)
    out50 = _stage_c(g1d)
    return out50.transpose(2, 0, 1)


# speedup vs baseline: 4.2412x; 1.2859x over previous
import functools

import jax
import jax.numpy as jnp
from jax import lax
from jax.experimental import pallas as pl
from jax.experimental.pallas import tpu as pltpu
from jax.experimental.pallas import tpu_sc as plsc

_B = 16384
_H = 50
_D = 32
_V = 1000000
_NW = 32
_NB = _B // 128
_NT_FULL = (_H // 8) * _NB
_NT_PART = _NB
_FLAT_N = _B * _H
_NBLK = _FLAT_N // 128
_MESH = plsc.VectorSubcoreMesh(core_axis_name="c", subcore_axis_name="s")


def _wid():
    return lax.axis_index("s") * 2 + lax.axis_index("c")


@functools.partial(
    pl.kernel,
    mesh=_MESH,
    out_type=jax.ShapeDtypeStruct((_FLAT_N,), jnp.int32),
    scratch_types=[
        pltpu.VMEM((8, 128), jnp.int32),
        pltpu.VMEM((2, 128), jnp.int32),
        pltpu.VMEM((1024,), jnp.int32),
    ],
    compiler_params=pltpu.CompilerParams(
        use_tc_tiling_on_sc=True, needs_layout_passes=False),
)
def _stage_a(inp2, flat, ibuf, pbuf, fbuf):
    w = _wid()

    def full_tile(ci, carry):
        t = ci * _NW + w
        gh = t // _NB
        cb = t % _NB
        pltpu.sync_copy(
            inp2.at[pl.ds(gh * 8, 8), pl.ds(cb * 128, 128)], ibuf)
        for k in range(8):
            for j in range(8):
                fbuf[pl.ds(k * 128 + j * 16, 16)] = ibuf[k, pl.ds(j * 16, 16)]
        pltpu.sync_copy(fbuf, flat.at[pl.ds(t * 1024, 1024)])
        return carry

    lax.fori_loop(0, _NT_FULL // _NW, full_tile, 0)

    def part_tile(ci, carry):
        t = ci * _NW + w
        pltpu.sync_copy(
            inp2.at[pl.ds(48, 2), pl.ds(t * 128, 128)], pbuf)
        for k in range(2):
            for j in range(8):
                fbuf[pl.ds(k * 128 + j * 16, 16)] = pbuf[k, pl.ds(j * 16, 16)]
        pltpu.sync_copy(
            fbuf.at[pl.ds(0, 256)],
            flat.at[pl.ds(_NT_FULL * 1024 + t * 256, 256)])
        return carry

    lax.fori_loop(0, _NT_PART // _NW, part_tile, 0)


@functools.partial(
    pl.kernel,
    mesh=_MESH,
    out_type=jax.ShapeDtypeStruct((_NBLK * 4128,), jnp.float32),
    scratch_types=[
        pltpu.VMEM((128,), jnp.int32),
        pltpu.VMEM((128,), jnp.int32),
        pltpu.VMEM((128, _D), jnp.float32),
        pltpu.VMEM((128, _D), jnp.float32),
        pltpu.VMEM((4128,), jnp.float32),
        pltpu.VMEM((4128,), jnp.float32),
        pltpu.SemaphoreType.DMA,
        pltpu.SemaphoreType.DMA,
        pltpu.SemaphoreType.DMA,
        pltpu.SemaphoreType.DMA,
        pltpu.SemaphoreType.DMA,
        pltpu.SemaphoreType.DMA,
    ],
    compiler_params=pltpu.CompilerParams(
        use_tc_tiling_on_sc=False, needs_layout_passes=False),
)
def _stage_b(flat, rv, g1d, idx0, idx1, gb0, gb1, ob0, ob1,
             is0, is1, gs0, gs1, os0, os1):
    w = _wid()
    idxb = [idx0, idx1]
    gbuf = [gb0, gb1]
    obuf = [ob0, ob1]
    isem = [is0, is1]
    gsem = [gs0, gs1]
    osem = [os0, os1]
    lane129 = lax.iota(jnp.int32, 16) * 129
    nblk = _NBLK // _NW

    def idx_wait(b, i):
        pltpu.make_async_copy(
            flat.at[pl.ds(i * 128, 128)], idxb[b], isem[b]).wait()

    def gather_start(b):
        pltpu.async_copy(rv.at[idxb[b]], gbuf[b], gsem[b])

    def gather_wait(b):
        pltpu.make_async_copy(
            rv.at[idxb[b]], gbuf[b], gsem[b]).wait()

    def transpose(b):
        for l in range(128):
            for dd in range(2):
                v = gbuf[b][l, pl.ds(dd * 16, 16)]
                plsc.store_scatter(
                    obuf[b], [lane129 + (dd * 2064 + l)], v)

    def out_start(b, i):
        pltpu.async_copy(obuf[b], g1d.at[pl.ds(i * 4128, 4128)], osem[b])

    def out_wait(b, i):
        pltpu.make_async_copy(
            obuf[b], g1d.at[pl.ds(i * 4128, 4128)], osem[b]).wait()

    for b in range(2):
        pltpu.async_copy(
            flat.at[pl.ds((b * _NW + w) * 128, 128)], idxb[b], isem[b])
    idx_wait(0, w)
    gather_start(0)

    def step(ci, b, pb):
        i_next = (ci + 1) * _NW + w
        i_cur = ci * _NW + w
        idx_wait(b, i_next)

        def _wait_ob():
            out_wait(b, (ci - 1) * _NW + w)

        if isinstance(ci, int):
            if ci >= 1:
                _wait_ob()
        else:
            pl.when(ci >= 1)(_wait_ob)

        gather_start(b)
        gather_wait(pb)

        def _prefetch():
            pltpu.async_copy(
                flat.at[pl.ds(((ci + 2) * _NW + w) * 128, 128)],
                idxb[pb], isem[pb])

        if isinstance(ci, int):
            if ci + 2 < nblk:
                _prefetch()
        else:
            pl.when(ci + 2 < nblk)(_prefetch)

        transpose(pb)
        out_start(pb, i_cur)

    def body(o, carry):
        for p in range(2):
            ci = o * 2 + p
            step(ci, (p + 1) % 2, p)
        return carry

    lax.fori_loop(0, (nblk - 2) // 2, body, 0)
    step(nblk - 2, (nblk - 1) % 2, (nblk - 2) % 2)

    lb = (nblk - 1) % 2
    li = (nblk - 1) * _NW + w
    gather_wait(lb)
    transpose(lb)
    out_start(lb, li)
    out_wait(lb, li)
    out_wait((nblk - 2) % 2, (nblk - 2) * _NW + w)


@functools.partial(
    pl.kernel,
    mesh=_MESH,
    out_type=jax.ShapeDtypeStruct((_H, _D, _B), jnp.float32),
    scratch_types=[
        pltpu.VMEM((4128,), jnp.float32),
        pltpu.VMEM((32, 128), jnp.float32),
    ],
    compiler_params=pltpu.CompilerParams(
        use_tc_tiling_on_sc=True, needs_layout_passes=False),
)
def _stage_c(g1d, out50, dbuf, cbuf):
    w = _wid()
    nblk = _NBLK // _NW

    def block(ci, carry):
        i = ci * _NW + w
        pltpu.sync_copy(g1d.at[pl.ds(i * 4128, 4128)], dbuf)
        for d in range(32):
            for j in range(8):
                cbuf[d, pl.ds(j * 16, 16)] = dbuf[pl.ds(d * 129 + j * 16, 16)]
        in_full = i < _NT_FULL * 8
        t = jnp.where(in_full, i // 8, 0)
        h_full = (t // _NB) * 8 + (i % 8)
        cb_full = t % _NB
        r = i - _NT_FULL * 8
        h_part = 48 + (r % 2)
        cb_part = r // 2
        h = jnp.where(in_full, h_full, h_part)
        cb = jnp.where(in_full, cb_full, cb_part)
        pltpu.sync_copy(cbuf, out50.at[h, :, pl.ds(cb * 128, 128)])
        return carry

    lax.fori_loop(0, nblk, block, 0)


def kernel(input, table):
    inp2 = input.T
    flat = _stage_a(inp2)
    g1d = _stage_b(flat, table)
    out50 = _stage_c(g1d)
    return out50.transpose(2, 0, 1)


# speedup vs baseline: 4.5243x; 1.0668x over previous
import functools

import jax
import jax.numpy as jnp
from jax import lax
from jax.experimental import pallas as pl
from jax.experimental.pallas import tpu as pltpu
from jax.experimental.pallas import tpu_sc as plsc

_B = 16384
_H = 50
_D = 32
_V = 1000000
_NW = 32
_NB = _B // 128
_NT_FULL = (_H // 8) * _NB
_NT_PART = _NB
_FLAT_N = _B * _H
_NBLK = _FLAT_N // 128
_MESH = plsc.VectorSubcoreMesh(core_axis_name="c", subcore_axis_name="s")


def _wid():
    return lax.axis_index("s") * 2 + lax.axis_index("c")


@functools.partial(
    pl.kernel,
    mesh=_MESH,
    out_type=jax.ShapeDtypeStruct((_FLAT_N,), jnp.int32),
    scratch_types=[
        pltpu.VMEM((8, 128), jnp.int32),
        pltpu.VMEM((2, 128), jnp.int32),
        pltpu.VMEM((1024,), jnp.int32),
    ],
    compiler_params=pltpu.CompilerParams(
        use_tc_tiling_on_sc=True, needs_layout_passes=False),
)
def _stage_a(inp2, flat, ibuf, pbuf, fbuf):
    w = _wid()

    def full_tile(ci, carry):
        t = ci * _NW + w
        gh = t // _NB
        cb = t % _NB
        pltpu.sync_copy(
            inp2.at[pl.ds(gh * 8, 8), pl.ds(cb * 128, 128)], ibuf)
        for k in range(8):
            for j in range(8):
                fbuf[pl.ds(k * 128 + j * 16, 16)] = ibuf[k, pl.ds(j * 16, 16)]
        pltpu.sync_copy(fbuf, flat.at[pl.ds(t * 1024, 1024)])
        return carry

    lax.fori_loop(0, _NT_FULL // _NW, full_tile, 0)

    def part_tile(ci, carry):
        t = ci * _NW + w
        pltpu.sync_copy(
            inp2.at[pl.ds(48, 2), pl.ds(t * 128, 128)], pbuf)
        for k in range(2):
            for j in range(8):
                fbuf[pl.ds(k * 128 + j * 16, 16)] = pbuf[k, pl.ds(j * 16, 16)]
        pltpu.sync_copy(
            fbuf.at[pl.ds(0, 256)],
            flat.at[pl.ds(_NT_FULL * 1024 + t * 256, 256)])
        return carry

    lax.fori_loop(0, _NT_PART // _NW, part_tile, 0)


@functools.partial(
    pl.kernel,
    mesh=_MESH,
    out_type=jax.ShapeDtypeStruct((_NBLK * 4128,), jnp.float32),
    scratch_types=[
        pltpu.VMEM((128,), jnp.int32),
        pltpu.VMEM((128,), jnp.int32),
        pltpu.VMEM((128, _D), jnp.float32),
        pltpu.VMEM((128, _D), jnp.float32),
        pltpu.VMEM((4128,), jnp.float32),
        pltpu.VMEM((4128,), jnp.float32),
        pltpu.SemaphoreType.DMA,
        pltpu.SemaphoreType.DMA,
        pltpu.SemaphoreType.DMA,
        pltpu.SemaphoreType.DMA,
        pltpu.SemaphoreType.DMA,
        pltpu.SemaphoreType.DMA,
    ],
    compiler_params=pltpu.CompilerParams(
        use_tc_tiling_on_sc=False, needs_layout_passes=False),
)
def _stage_b(flat, rv, g1d, idx0, idx1, gb0, gb1, ob0, ob1,
             is0, is1, gs0, gs1, os0, os1):
    w = _wid()
    idxb = [idx0, idx1]
    gbuf = [gb0, gb1]
    obuf = [ob0, ob1]
    isem = [is0, is1]
    gsem = [gs0, gs1]
    osem = [os0, os1]
    lane129 = lax.iota(jnp.int32, 16) * 129
    nblk = _NBLK // _NW

    def idx_wait(b, i):
        pltpu.make_async_copy(
            flat.at[pl.ds(i * 128, 128)], idxb[b], isem[b]).wait()

    def gather_start(b):
        pltpu.async_copy(rv.at[idxb[b]], gbuf[b], gsem[b])

    def gather_wait(b):
        pltpu.make_async_copy(
            rv.at[idxb[b]], gbuf[b], gsem[b]).wait()

    def transpose(b):
        for l in range(128):
            for dd in range(2):
                v = gbuf[b][l, pl.ds(dd * 16, 16)]
                plsc.store_scatter(
                    obuf[b], [lane129 + (dd * 2064 + l)], v)

    def out_start(b, i):
        pltpu.async_copy(obuf[b], g1d.at[pl.ds(i * 4128, 4128)], osem[b])

    def out_wait(b, i):
        pltpu.make_async_copy(
            obuf[b], g1d.at[pl.ds(i * 4128, 4128)], osem[b]).wait()

    for b in range(2):
        pltpu.async_copy(
            flat.at[pl.ds((b * _NW + w) * 128, 128)], idxb[b], isem[b])
    idx_wait(0, w)
    gather_start(0)

    def step(ci, b, pb):
        i_next = (ci + 1) * _NW + w
        i_cur = ci * _NW + w
        idx_wait(b, i_next)

        def _wait_ob():
            out_wait(b, (ci - 1) * _NW + w)

        if isinstance(ci, int):
            if ci >= 1:
                _wait_ob()
        else:
            pl.when(ci >= 1)(_wait_ob)

        gather_start(b)
        gather_wait(pb)

        def _prefetch():
            pltpu.async_copy(
                flat.at[pl.ds(((ci + 2) * _NW + w) * 128, 128)],
                idxb[pb], isem[pb])

        if isinstance(ci, int):
            if ci + 2 < nblk:
                _prefetch()
        else:
            pl.when(ci + 2 < nblk)(_prefetch)

        transpose(pb)
        out_start(pb, i_cur)

    def body(o, carry):
        for p in range(2):
            ci = o * 2 + p
            step(ci, (p + 1) % 2, p)
        return carry

    lax.fori_loop(0, (nblk - 2) // 2, body, 0)
    step(nblk - 2, (nblk - 1) % 2, (nblk - 2) % 2)

    lb = (nblk - 1) % 2
    li = (nblk - 1) * _NW + w
    gather_wait(lb)
    transpose(lb)
    out_start(lb, li)
    out_wait(lb, li)
    out_wait((nblk - 2) % 2, (nblk - 2) * _NW + w)


@functools.partial(
    pl.kernel,
    mesh=_MESH,
    out_type=jax.ShapeDtypeStruct((_H, _D, _B), jnp.float32),
    scratch_types=[
        pltpu.VMEM((4128,), jnp.float32),
        pltpu.VMEM((4128,), jnp.float32),
        pltpu.VMEM((32, 128), jnp.float32),
        pltpu.VMEM((32, 128), jnp.float32),
        pltpu.SemaphoreType.DMA,
        pltpu.SemaphoreType.DMA,
        pltpu.SemaphoreType.DMA,
        pltpu.SemaphoreType.DMA,
    ],
    compiler_params=pltpu.CompilerParams(
        use_tc_tiling_on_sc=True, needs_layout_passes=False),
)
def _stage_c(g1d, out50, db0, db1, cb0, cb1, ds0, ds1, os0, os1):
    w = _wid()
    dbuf = [db0, db1]
    cbuf = [cb0, cb1]
    dsem = [ds0, ds1]
    osem = [os0, os1]
    nblk = _NBLK // _NW

    def in_start(b, i):
        pltpu.async_copy(g1d.at[pl.ds(i * 4128, 4128)], dbuf[b], dsem[b])

    def in_wait(b, i):
        pltpu.make_async_copy(
            g1d.at[pl.ds(i * 4128, 4128)], dbuf[b], dsem[b]).wait()

    def decode(i):
        in_full = i < _NT_FULL * 8
        t = jnp.where(in_full, i // 8, 0)
        h_full = (t // _NB) * 8 + (i % 8)
        cb_full = t % _NB
        r = i - _NT_FULL * 8
        h_part = 48 + (r % 2)
        cb_part = r // 2
        h = jnp.where(in_full, h_full, h_part)
        cb = jnp.where(in_full, cb_full, cb_part)
        return h, cb

    def out_start(b, i):
        h, cb = decode(i)
        pltpu.async_copy(
            cbuf[b], out50.at[h, :, pl.ds(cb * 128, 128)], osem[b])

    def out_wait(b, i):
        h, cb = decode(i)
        pltpu.make_async_copy(
            cbuf[b], out50.at[h, :, pl.ds(cb * 128, 128)], osem[b]).wait()

    def repack(b):
        for d in range(32):
            for j in range(8):
                cbuf[b][d, pl.ds(j * 16, 16)] = \
                    dbuf[b][pl.ds(d * 129 + j * 16, 16)]

    in_start(0, w)

    def step(ci, b, nb):
        i = ci * _NW + w
        in_wait(b, i)

        def _next_in():
            in_start(nb, (ci + 1) * _NW + w)

        if isinstance(ci, int):
            if ci + 1 < nblk:
                _next_in()
        else:
            pl.when(ci + 1 < nblk)(_next_in)

        def _wait_out():
            out_wait(b, (ci - 2) * _NW + w)

        if isinstance(ci, int):
            if ci >= 2:
                _wait_out()
        else:
            pl.when(ci >= 2)(_wait_out)

        repack(b)
        out_start(b, i)

    def body(o, carry):
        for p in range(2):
            ci = o * 2 + p
            step(ci, p, (p + 1) % 2)
        return carry

    lax.fori_loop(0, nblk // 2, body, 0)
    out_wait(0, (nblk - 2) * _NW + w)
    out_wait(1, (nblk - 1) * _NW + w)


def kernel(input, table):
    inp2 = input.T
    flat = _stage_a(inp2)
    g1d = _stage_b(flat, table)
    out50 = _stage_c(g1d)
    return out50.transpose(2, 0, 1)
